# baseline (speedup 1.0000x reference)
import functools

import jax
import jax.numpy as jnp
from jax import lax
from jax.experimental import pallas as pl
from jax.experimental.pallas import tpu as pltpu
from jax.experimental.pallas import tpu_sc as plsc

VOCAB = 37
EMBED = 16
BATCH = 16384
SEQ = 200
N = BATCH * SEQ

NUM_CORES = 2
NUM_SUBCORES = 16
NW = NUM_CORES * NUM_SUBCORES
ROWS_W = BATCH // NW
CB = 8
CHUNK = CB * SEQ
STEPS = ROWS_W // CB
NBUF = 2
GROUPS = CHUNK // 16

_mesh = plsc.VectorSubcoreMesh(core_axis_name="c", subcore_axis_name="s")


@functools.partial(
    pl.kernel,
    mesh=_mesh,
    out_type=jax.ShapeDtypeStruct((BATCH, EMBED, SEQ), jnp.float32),
    scratch_types=[
        pltpu.VMEM((VOCAB * EMBED,), jnp.float32),
        pltpu.VMEM((NBUF, CHUNK), jnp.int32),
        pltpu.VMEM((NBUF, CB, EMBED, SEQ), jnp.float32),
        [pltpu.SemaphoreType.DMA] * NBUF,
        [pltpu.SemaphoreType.DMA] * NBUF,
        pltpu.SemaphoreType.DMA,
    ],
    compiler_params=pltpu.CompilerParams(use_tc_tiling_on_sc=False,
                                         needs_layout_passes=False,
                                         disable_bounds_checks=True),
)
def _embed_lookup(ids, table_hbm, out, tbl_v, idx_v, rows_v, isems, osems,
                  tsem):
    wid = lax.axis_index("s") * NUM_CORES + lax.axis_index("c")
    row_base = wid * ROWS_W
    id_base = row_base * SEQ

    pltpu.async_copy(table_hbm, tbl_v, tsem)
    for b in range(NBUF):
        pltpu.async_copy(ids.at[pl.ds(id_base + b * CHUNK, CHUNK)],
                         idx_v.at[b], isems[b])
    pltpu.make_async_copy(table_hbm, tbl_v, tsem).wait()

    iota16 = lax.iota(jnp.int32, 16)

    def compute_chunk(b):
        def group(g, carry):
            f = g * 16 + iota16
            bb = f // SEQ
            s = f - bb * SEQ
            ids_v = idx_v.at[b][pl.ds(g * 16, 16)]
            gbase = ids_v * EMBED
            for d in range(EMBED):
                v = plsc.load_gather(tbl_v, [gbase + d])
                plsc.store_scatter(rows_v.at[b],
                                   [bb, jnp.full((16,), d, jnp.int32), s], v)
            return carry
        lax.fori_loop(0, GROUPS, group, 0)

    def process(step, b, refill):
        pltpu.make_async_copy(ids.at[pl.ds(0, CHUNK)], idx_v.at[b],
                              isems[b]).wait()
        @pl.when(step >= NBUF)
        def _():
            pltpu.make_async_copy(rows_v.at[b],
                                  out.at[pl.ds(0, CB)], osems[b]).wait()
        compute_chunk(b)
        pltpu.async_copy(rows_v.at[b],
                         out.at[pl.ds(row_base + step * CB, CB)], osems[b])
        if refill:
            pltpu.async_copy(
                ids.at[pl.ds(id_base + (step + NBUF) * CHUNK, CHUNK)],
                idx_v.at[b], isems[b])

    num_groups = (STEPS - NBUF) // NBUF

    def outer(i, carry):
        for b in range(NBUF):
            process(i * NBUF + b, b, refill=True)
        return carry

    lax.fori_loop(0, num_groups, outer, 0)

    for step in range(num_groups * NBUF, STEPS):
        process(step, step % NBUF, refill=step + NBUF < STEPS)

    for b in range(NBUF):
        pltpu.make_async_copy(rows_v.at[b], out.at[pl.ds(0, CB)],
                              osems[b]).wait()


def kernel(char_ids, embed_weight):
    ids = char_ids.reshape(N).astype(jnp.int32)
    out_bes = _embed_lookup(ids, embed_weight.reshape(VOCAB * EMBED))
    return jnp.transpose(out_bes, (0, 2, 1))


# speedup vs baseline: 1.5171x; 1.5171x over previous
import functools

import jax
import jax.numpy as jnp
from jax import lax
from jax.experimental import pallas as pl
from jax.experimental.pallas import tpu as pltpu
from jax.experimental.pallas import tpu_sc as plsc

VOCAB = 37
EMBED = 16
BATCH = 16384
SEQ = 200
N = BATCH * SEQ

NUM_CORES = 2
NUM_SUBCORES = 16
NW = NUM_CORES * NUM_SUBCORES
ROWS_W = BATCH // NW
CB = 8
CHUNK = CB * SEQ
STEPS = ROWS_W // CB
NBUF = 2
GROUPS = CHUNK // 16

_mesh = plsc.VectorSubcoreMesh(core_axis_name="c", subcore_axis_name="s")


@functools.partial(
    pl.kernel,
    mesh=_mesh,
    out_type=jax.ShapeDtypeStruct((BATCH, EMBED, SEQ), jnp.float32),
    scratch_types=[
        pltpu.VMEM((VOCAB * EMBED,), jnp.float32),
        pltpu.VMEM((NBUF, CHUNK), jnp.int32),
        pltpu.VMEM((NBUF, CB, EMBED, SEQ), jnp.float32),
        [pltpu.SemaphoreType.DMA] * NBUF,
        [pltpu.SemaphoreType.DMA] * NBUF,
        pltpu.SemaphoreType.DMA,
    ],
    compiler_params=pltpu.CompilerParams(use_tc_tiling_on_sc=False,
                                         needs_layout_passes=False,
                                         disable_bounds_checks=True),
)
def _embed_lookup(ids, table_hbm, out, tbl_v, idx_v, rows_v, isems, osems,
                  tsem):
    wid = lax.axis_index("s") * NUM_CORES + lax.axis_index("c")
    row_base = wid * ROWS_W
    id_base = row_base * SEQ

    pltpu.async_copy(table_hbm, tbl_v, tsem)
    for b in range(NBUF):
        pltpu.async_copy(ids.at[pl.ds(id_base + b * CHUNK, CHUNK)],
                         idx_v.at[b], isems[b])
    pltpu.make_async_copy(table_hbm, tbl_v, tsem).wait()

    iota16 = lax.iota(jnp.int32, 16)

    def compute_chunk(b):
        def group(g, carry):
            f = g * 16 + iota16
            bb = f // SEQ
            s = f - bb * SEQ
            ids_v = idx_v.at[b][pl.ds(g * 16, 16)]
            gbase = ids_v * EMBED
            vals = [plsc.load_gather(tbl_v, [gbase + d]) for d in range(EMBED)]
            for d in range(EMBED):
                plsc.store_scatter(rows_v.at[b],
                                   [bb, jnp.full((16,), d, jnp.int32), s],
                                   vals[d])
            return carry
        lax.fori_loop(0, GROUPS, group, 0)

    def process(step, b, refill):
        pltpu.make_async_copy(ids.at[pl.ds(0, CHUNK)], idx_v.at[b],
                              isems[b]).wait()
        @pl.when(step >= NBUF)
        def _():
            pltpu.make_async_copy(rows_v.at[b],
                                  out.at[pl.ds(0, CB)], osems[b]).wait()
        compute_chunk(b)
        pltpu.async_copy(rows_v.at[b],
                         out.at[pl.ds(row_base + step * CB, CB)], osems[b])
        if refill:
            pltpu.async_copy(
                ids.at[pl.ds(id_base + (step + NBUF) * CHUNK, CHUNK)],
                idx_v.at[b], isems[b])

    num_groups = (STEPS - NBUF) // NBUF

    def outer(i, carry):
        for b in range(NBUF):
            process(i * NBUF + b, b, refill=True)
        return carry

    lax.fori_loop(0, num_groups, outer, 0)

    for step in range(num_groups * NBUF, STEPS):
        process(step, step % NBUF, refill=step + NBUF < STEPS)

    for b in range(NBUF):
        pltpu.make_async_copy(rows_v.at[b], out.at[pl.ds(0, CB)],
                              osems[b]).wait()


def kernel(char_ids, embed_weight):
    ids = char_ids.reshape(N).astype(jnp.int32)
    out_bes = _embed_lookup(ids, embed_weight.reshape(VOCAB * EMBED))
    return jnp.transpose(out_bes, (0, 2, 1))


# speedup vs baseline: 2.5180x; 1.6598x over previous
import functools

import jax
import jax.numpy as jnp
from jax import lax
from jax.experimental import pallas as pl
from jax.experimental.pallas import tpu as pltpu
from jax.experimental.pallas import tpu_sc as plsc

VOCAB = 37
EMBED = 16
BATCH = 16384
SEQ = 200
N = BATCH * SEQ

NUM_CORES = 2
NUM_SUBCORES = 16
NW = NUM_CORES * NUM_SUBCORES
ROWS_W = BATCH // NW
CB = 2
CHUNK = CB * SEQ
STEPS = ROWS_W // CB
NBUF = 2
GROUPS = CHUNK // 16

_mesh = plsc.VectorSubcoreMesh(core_axis_name="c", subcore_axis_name="s")


@functools.partial(
    pl.kernel,
    mesh=_mesh,
    out_type=jax.ShapeDtypeStruct((BATCH, EMBED, SEQ), jnp.float32),
    scratch_types=[
        pltpu.VMEM((VOCAB * EMBED,), jnp.float32),
        [pltpu.VMEM((CHUNK,), jnp.int32)] * NBUF,
        [pltpu.VMEM((CB, EMBED, SEQ), jnp.float32)] * NBUF,
        [pltpu.SemaphoreType.DMA] * NBUF,
        [pltpu.SemaphoreType.DMA] * NBUF,
        pltpu.SemaphoreType.DMA,
    ],
    compiler_params=pltpu.CompilerParams(use_tc_tiling_on_sc=True,
                                         needs_layout_passes=False,
                                         disable_bounds_checks=True),
)
def _embed_lookup(ids, table_hbm, out, tbl_v, idx_v, rows_v, isems, osems,
                  tsem):
    wid = lax.axis_index("s") * NUM_CORES + lax.axis_index("c")
    row_base = wid * ROWS_W
    id_base = row_base * SEQ

    pltpu.async_copy(table_hbm, tbl_v, tsem)
    for b in range(NBUF):
        pltpu.async_copy(ids.at[pl.ds(id_base + b * CHUNK, CHUNK)],
                         idx_v[b], isems[b])
    pltpu.make_async_copy(table_hbm, tbl_v, tsem).wait()

    iota16 = lax.iota(jnp.int32, 16)

    def compute_chunk(b):
        def group(g, carry):
            f = g * 16 + iota16
            bb = f // SEQ
            s = f - bb * SEQ
            ids_v = idx_v[b][pl.ds(g * 16, 16)]
            gbase = ids_v * EMBED
            vals = [plsc.load_gather(tbl_v, [gbase + d]) for d in range(EMBED)]
            for d in range(EMBED):
                plsc.store_scatter(rows_v[b],
                                   [bb, jnp.full((16,), d, jnp.int32), s],
                                   vals[d])
            return carry
        lax.fori_loop(0, GROUPS, group, 0)

    def process(step, b, refill):
        pltpu.make_async_copy(ids.at[pl.ds(0, CHUNK)], idx_v[b],
                              isems[b]).wait()
        @pl.when(step >= NBUF)
        def _():
            pltpu.make_async_copy(rows_v[b],
                                  out.at[pl.ds(0, CB)], osems[b]).wait()
        compute_chunk(b)
        pltpu.async_copy(rows_v[b],
                         out.at[pl.ds(row_base + step * CB, CB)], osems[b])
        if refill:
            pltpu.async_copy(
                ids.at[pl.ds(id_base + (step + NBUF) * CHUNK, CHUNK)],
                idx_v[b], isems[b])

    num_groups = (STEPS - NBUF) // NBUF

    def outer(i, carry):
        for b in range(NBUF):
            process(i * NBUF + b, b, refill=True)
        return carry

    lax.fori_loop(0, num_groups, outer, 0)

    for step in range(num_groups * NBUF, STEPS):
        process(step, step % NBUF, refill=step + NBUF < STEPS)

    for b in range(NBUF):
        pltpu.make_async_copy(rows_v[b], out.at[pl.ds(0, CB)],
                              osems[b]).wait()


def kernel(char_ids, embed_weight):
    ids = char_ids.reshape(N).astype(jnp.int32)
    out_bes = _embed_lookup(ids, embed_weight.reshape(VOCAB * EMBED))
    return jnp.transpose(out_bes, (0, 2, 1))


# speedup vs baseline: 3.3722x; 1.3392x over previous
import functools

import jax
import jax.numpy as jnp
from jax import lax
from jax.experimental import pallas as pl
from jax.experimental.pallas import tpu as pltpu
from jax.experimental.pallas import tpu_sc as plsc

VOCAB = 37
EMBED = 16
BATCH = 16384
SEQ = 200

NUM_CORES = 2
NUM_SUBCORES = 16
NW = NUM_CORES * NUM_SUBCORES
BW = BATCH // NW
CS = 8
CBL = 256
BSUB = BW // CBL
STEPS = (SEQ // CS) * BSUB
NBUF = 2
VPAD = 48

_mesh = plsc.VectorSubcoreMesh(core_axis_name="c", subcore_axis_name="s")


@functools.partial(
    pl.kernel,
    mesh=_mesh,
    out_type=jax.ShapeDtypeStruct((SEQ, EMBED, BATCH), jnp.float32),
    scratch_types=[
        pltpu.VMEM((VPAD * EMBED,), jnp.float32),
        [pltpu.VMEM((VPAD,), jnp.float32)] * EMBED,
        [pltpu.VMEM((CS, CBL), jnp.int32)] * NBUF,
        [pltpu.VMEM((CS, EMBED, CBL), jnp.float32)] * NBUF,
        [pltpu.SemaphoreType.DMA] * NBUF,
        [pltpu.SemaphoreType.DMA] * NBUF,
        pltpu.SemaphoreType.DMA,
    ],
    compiler_params=pltpu.CompilerParams(use_tc_tiling_on_sc=True,
                                         needs_layout_passes=False,
                                         disable_bounds_checks=True),
)
def _embed_lookup(ids, table_hbm, out, tbl_v, tbl_e, idx_v, rows_v, isems,
                  osems, tsem):
    wid = lax.axis_index("s") * NUM_CORES + lax.axis_index("c")
    b_base = wid * BW

    iota16 = lax.iota(jnp.int32, 16)

    def slices(step):
        st = step // BSUB
        bs = step - st * BSUB
        return st * CS, b_base + bs * CBL

    pltpu.async_copy(table_hbm, tbl_v.at[pl.ds(0, VOCAB * EMBED)], tsem)
    for b in range(NBUF):
        s0, b0 = slices(b)
        pltpu.async_copy(ids.at[pl.ds(s0, CS), pl.ds(b0, CBL)], idx_v[b],
                         isems[b])
    pltpu.make_async_copy(table_hbm, tbl_v.at[pl.ds(0, VOCAB * EMBED)],
                          tsem).wait()

    for e in range(EMBED):
        for k in range(VPAD // 16):
            v = plsc.load_gather(tbl_v, [(iota16 + 16 * k) * EMBED + e])
            tbl_e[e][pl.ds(16 * k, 16)] = v

    def compute_chunk(b):
        def per_seq(si, carry):
            for bg in range(CBL // 16):
                ids_v = idx_v[b][si, pl.ds(bg * 16, 16)]
                for e in range(EMBED):
                    rows_v[b][si, e, pl.ds(bg * 16, 16)] = (
                        plsc.load_gather(tbl_e[e], [ids_v]))
            return carry
        lax.fori_loop(0, CS, per_seq, 0)

    def process(step, b, refill):
        pltpu.make_async_copy(ids.at[pl.ds(0, CS), pl.ds(0, CBL)], idx_v[b],
                              isems[b]).wait()
        @pl.when(step >= NBUF)
        def _():
            pltpu.make_async_copy(
                rows_v[b], out.at[pl.ds(0, CS), :, pl.ds(0, CBL)],
                osems[b]).wait()
        compute_chunk(b)
        s0, b0 = slices(step)
        pltpu.async_copy(rows_v[b], out.at[pl.ds(s0, CS), :, pl.ds(b0, CBL)],
                         osems[b])
        if refill:
            s1, b1 = slices(step + NBUF)
            pltpu.async_copy(ids.at[pl.ds(s1, CS), pl.ds(b1, CBL)], idx_v[b],
                             isems[b])

    num_groups = (STEPS - NBUF) // NBUF

    def outer(i, carry):
        for b in range(NBUF):
            process(i * NBUF + b, b, refill=True)
        return carry

    lax.fori_loop(0, num_groups, outer, 0)

    for step in range(num_groups * NBUF, STEPS):
        process(step, step % NBUF, refill=step + NBUF < STEPS)

    for b in range(NBUF):
        pltpu.make_async_copy(rows_v[b],
                              out.at[pl.ds(0, CS), :, pl.ds(0, CBL)],
                              osems[b]).wait()


def kernel(char_ids, embed_weight):
    ids_sb = jnp.transpose(char_ids, (1, 0)).astype(jnp.int32)
    out_seb = _embed_lookup(ids_sb, embed_weight.reshape(VOCAB * EMBED))
    return jnp.transpose(out_seb, (2, 0, 1))
